# Initial kernel scaffold; baseline (speedup 1.0000x reference)
import jax
import jax.numpy as jnp
from jax.experimental import pallas as pl


def kernel(features, edge_index, W1, b1, W2, b2):
    raise NotImplementedError("write your pallas kernel here")


# speedup vs baseline: 7.9848x; 7.9848x over previous
import functools

import jax
import jax.numpy as jnp
from jax import lax
from jax.experimental import pallas as pl
from jax.experimental.pallas import tpu as pltpu
from jax.experimental.pallas import tpu_sc as plsc

N = 10000
E = 320000
D_IN = 128
D_H = 256
D_OUT = 128

NCORE = 2
NSUB = 16
NW = NCORE * NSUB
EPW = E // NW
NB = 80
BB = EPW // NB
RPT = N // NSUB
BN = 1000

_sc_mesh = plsc.VectorSubcoreMesh(core_axis_name="c", subcore_axis_name="s")


ROWS_A = 624
ROWS_LAST = N - ROWS_A * (NSUB - 1)


def _sliced_rows(s, fn):
    @pl.when(s < NSUB - 1)
    def _():
        fn(pl.multiple_of(s * ROWS_A, 8), ROWS_A)

    @pl.when(s == NSUB - 1)
    def _():
        fn(ROWS_A * (NSUB - 1), ROWS_LAST)


@functools.partial(
    pl.kernel,
    mesh=_sc_mesh,
    out_type=jax.ShapeDtypeStruct((NCORE, 2, N, 128), jnp.float32),
    scratch_types=[
        pltpu.VMEM((NB, BB), jnp.int32),
        pltpu.VMEM((BB, 128), jnp.float32),
        pltpu.VMEM_SHARED((N, 128), jnp.float32),
    ],
)
def _deg_kernel(src_hbm, dst_hbm, ones_hbm, zmat_hbm, out_hbm,
                idx_v, ones_v, hist):
    c = lax.axis_index("c")
    s = lax.axis_index("s")
    w = c * NSUB + s
    pltpu.sync_copy(ones_hbm, ones_v)
    for which, idx_hbm in enumerate((src_hbm, dst_hbm)):
        _sliced_rows(s, lambda o, r: pltpu.sync_copy(
            zmat_hbm.at[pl.ds(o, r)], hist.at[pl.ds(o, r)]))
        pltpu.sync_copy(idx_hbm.at[w], idx_v)
        plsc.subcore_barrier()

        def body(j, _):
            pltpu.sync_copy(ones_v, hist.at[idx_v.at[j]], add=True)
            return 0

        lax.fori_loop(0, NB, body, 0)
        plsc.subcore_barrier()
        _sliced_rows(s, lambda o, r, which=which: pltpu.sync_copy(
            hist.at[pl.ds(o, r)], out_hbm.at[c, which, pl.ds(o, r)]))
        plsc.subcore_barrier()


@functools.partial(
    pl.kernel,
    mesh=_sc_mesh,
    out_type=jax.ShapeDtypeStruct((NCORE, N, 128), jnp.float32),
    scratch_types=[
        pltpu.VMEM((NB, BB), jnp.int32),
        pltpu.VMEM((NB, BB), jnp.int32),
        pltpu.VMEM((BB, 128), jnp.float32),
        pltpu.VMEM_SHARED((N, 128), jnp.float32),
        pltpu.SemaphoreType.DMA,
    ],
)
def _spmm_kernel(h_hbm, src_hbm, dst_hbm, zmat_hbm, out_hbm,
                 src_v, dst_v, gbuf, agg_sh, sem):
    c = lax.axis_index("c")
    s = lax.axis_index("s")
    w = c * NSUB + s
    _sliced_rows(s, lambda o, r: pltpu.sync_copy(
        zmat_hbm.at[pl.ds(o, r)], agg_sh.at[pl.ds(o, r)]))
    pltpu.sync_copy(src_hbm.at[w], src_v)
    pltpu.sync_copy(dst_hbm.at[w], dst_v)
    plsc.subcore_barrier()

    def body(j, _):
        pltpu.async_copy(h_hbm.at[src_v.at[j]], gbuf, sem).wait()
        pltpu.sync_copy(gbuf, agg_sh.at[dst_v.at[j]], add=True)
        return 0

    lax.fori_loop(0, NB, body, 0)
    plsc.subcore_barrier()
    _sliced_rows(s, lambda o, r: pltpu.sync_copy(
        agg_sh.at[pl.ds(o, r)], out_hbm.at[c, pl.ds(o, r)]))


def _prep_body(hist_ref, x_ref, xs_ref, ns_ref, nd_ref):
    h = hist_ref[...]
    deg_o = (h[0, 0] + h[1, 0])[:, 0:1]
    deg_i = (h[0, 1] + h[1, 1])[:, 0:1]
    ns = lax.rsqrt(jnp.maximum(deg_o, 1.0))
    nd = lax.rsqrt(jnp.maximum(deg_i, 1.0))
    xs_ref[...] = x_ref[...] * ns
    ns_ref[...] = ns
    nd_ref[...] = nd


def _prep_call(hist4, x):
    return pl.pallas_call(
        _prep_body,
        grid=(N // BN,),
        in_specs=[
            pl.BlockSpec((NCORE, 2, BN, 128), lambda i: (0, 0, i, 0)),
            pl.BlockSpec((BN, D_IN), lambda i: (i, 0)),
        ],
        out_specs=[
            pl.BlockSpec((BN, D_IN), lambda i: (i, 0)),
            pl.BlockSpec((BN, 1), lambda i: (i, 0)),
            pl.BlockSpec((BN, 1), lambda i: (i, 0)),
        ],
        out_shape=[
            jax.ShapeDtypeStruct((N, D_IN), jnp.float32),
            jax.ShapeDtypeStruct((N, 1), jnp.float32),
            jax.ShapeDtypeStruct((N, 1), jnp.float32),
        ],
    )(hist4, x)


def _mid_body(p_ref, w1_ref, b1_ref, ns_ref, nd_ref, w2_ref, h2_ref):
    p = p_ref[0] + p_ref[1]
    a = jnp.dot(p, w1_ref[...], preferred_element_type=jnp.float32)
    o1 = a * nd_ref[...] + b1_ref[...]
    r = jnp.maximum(o1, 0.0) * ns_ref[...]
    h2_ref[...] = jnp.dot(r, w2_ref[...], preferred_element_type=jnp.float32)


def _mid_call(p, W1, b1r, ns, nd, W2):
    return pl.pallas_call(
        _mid_body,
        grid=(N // BN,),
        in_specs=[
            pl.BlockSpec((NCORE, BN, 128), lambda i: (0, i, 0)),
            pl.BlockSpec((D_IN, D_H), lambda i: (0, 0)),
            pl.BlockSpec((1, D_H), lambda i: (0, 0)),
            pl.BlockSpec((BN, 1), lambda i: (i, 0)),
            pl.BlockSpec((BN, 1), lambda i: (i, 0)),
            pl.BlockSpec((D_H, D_OUT), lambda i: (0, 0)),
        ],
        out_specs=pl.BlockSpec((BN, D_OUT), lambda i: (i, 0)),
        out_shape=jax.ShapeDtypeStruct((N, D_OUT), jnp.float32),
    )(p, W1, b1r, ns, nd, W2)


def _fin_body(q_ref, nd_ref, b2_ref, o_ref):
    o_ref[...] = (q_ref[0] + q_ref[1]) * nd_ref[...] + b2_ref[...]


def _fin_call(q, nd, b2r):
    return pl.pallas_call(
        _fin_body,
        grid=(N // BN,),
        in_specs=[
            pl.BlockSpec((NCORE, BN, D_OUT), lambda i: (0, i, 0)),
            pl.BlockSpec((BN, 1), lambda i: (i, 0)),
            pl.BlockSpec((1, D_OUT), lambda i: (0, 0)),
        ],
        out_specs=pl.BlockSpec((BN, D_OUT), lambda i: (i, 0)),
        out_shape=jax.ShapeDtypeStruct((N, D_OUT), jnp.float32),
    )(q, nd, b2r)


def kernel(features, edge_index, W1, b1, W2, b2):
    ei = edge_index.astype(jnp.int32)
    src4 = ei[0].reshape(NW, NB, BB)
    dst4 = ei[1].reshape(NW, NB, BB)
    ones128 = jnp.ones((BB, 128), jnp.float32)
    zmat = jnp.zeros((N, 128), jnp.float32)

    hist4 = _deg_kernel(src4, dst4, ones128, zmat)
    xs, ns, nd = _prep_call(hist4, features)
    p = _spmm_kernel(xs, src4, dst4, zmat)
    h2 = _mid_call(p, W1, b1.reshape(1, D_H), ns, nd, W2)
    q = _spmm_kernel(h2, src4, dst4, zmat)
    return _fin_call(q, nd, b2.reshape(1, D_OUT))


# speedup vs baseline: 9.4306x; 1.1811x over previous
import functools

import jax
import jax.numpy as jnp
from jax import lax
from jax.experimental import pallas as pl
from jax.experimental.pallas import tpu as pltpu
from jax.experimental.pallas import tpu_sc as plsc

N = 10000
E = 320000
D_IN = 128
D_H = 256
D_OUT = 128

NCORE = 2
NSUB = 16
NW = NCORE * NSUB
EPW = E // NW
NB = 80
BB = EPW // NB
NBH = NB // 2
RPT = N // NSUB
BN = 1000

_sc_mesh = plsc.VectorSubcoreMesh(core_axis_name="c", subcore_axis_name="s")


ROWS_A = 624
ROWS_LAST = N - ROWS_A * (NSUB - 1)


def _sliced_rows(s, fn):
    @pl.when(s < NSUB - 1)
    def _():
        fn(pl.multiple_of(s * ROWS_A, 8), ROWS_A)

    @pl.when(s == NSUB - 1)
    def _():
        fn(ROWS_A * (NSUB - 1), ROWS_LAST)


@functools.partial(
    pl.kernel,
    mesh=_sc_mesh,
    out_type=jax.ShapeDtypeStruct((NCORE, 2, N, 128), jnp.float32),
    scratch_types=[
        pltpu.VMEM((NB, BB), jnp.int32),
        pltpu.VMEM((BB, 128), jnp.float32),
        pltpu.VMEM_SHARED((N, 128), jnp.float32),
    ],
)
def _deg_kernel(src_hbm, dst_hbm, ones_hbm, zmat_hbm, out_hbm,
                idx_v, ones_v, hist):
    c = lax.axis_index("c")
    s = lax.axis_index("s")
    w = c * NSUB + s
    pltpu.sync_copy(ones_hbm, ones_v)
    for which, idx_hbm in enumerate((src_hbm, dst_hbm)):
        _sliced_rows(s, lambda o, r: pltpu.sync_copy(
            zmat_hbm.at[pl.ds(o, r)], hist.at[pl.ds(o, r)]))
        pltpu.sync_copy(idx_hbm.at[w], idx_v)
        plsc.subcore_barrier()

        def body(j, _):
            pltpu.sync_copy(ones_v, hist.at[idx_v.at[j]], add=True)
            return 0

        lax.fori_loop(0, NB, body, 0)
        plsc.subcore_barrier()
        _sliced_rows(s, lambda o, r, which=which: pltpu.sync_copy(
            hist.at[pl.ds(o, r)], out_hbm.at[c, which, pl.ds(o, r)]))
        plsc.subcore_barrier()


@functools.partial(
    pl.kernel,
    mesh=_sc_mesh,
    out_type=jax.ShapeDtypeStruct((NCORE, N, 128), jnp.float32),
    scratch_types=[
        pltpu.VMEM((NBH, BB), jnp.int32),
        pltpu.VMEM((NBH, BB), jnp.int32),
        pltpu.VMEM((BB, 128), jnp.float32),
        pltpu.VMEM((BB, 128), jnp.float32),
        pltpu.VMEM_SHARED((N, 128), jnp.float32),
        pltpu.SemaphoreType.DMA,
        pltpu.SemaphoreType.DMA,
    ],
)
def _spmm_kernel(h_hbm, src_hbm, dst_hbm, zmat_hbm, out_hbm,
                 src_v, dst_v, gbuf0, gbuf1, agg_sh, sem0, sem1):
    c = lax.axis_index("c")
    s = lax.axis_index("s")
    w = c * NSUB + s
    _sliced_rows(s, lambda o, r: pltpu.sync_copy(
        zmat_hbm.at[pl.ds(o, r)], agg_sh.at[pl.ds(o, r)]))
    plsc.subcore_barrier()

    for h in range(2):
        pltpu.sync_copy(src_hbm.at[w, pl.ds(h * NBH, NBH)], src_v)
        pltpu.sync_copy(dst_hbm.at[w, pl.ds(h * NBH, NBH)], dst_v)
        pltpu.async_copy(h_hbm.at[src_v.at[0]], gbuf0, sem0)

        def body(jj, _):
            j = 2 * jj
            pltpu.make_async_copy(h_hbm.at[src_v.at[j]], gbuf0, sem0).wait()
            pltpu.async_copy(h_hbm.at[src_v.at[j + 1]], gbuf1, sem1)
            pltpu.sync_copy(gbuf0, agg_sh.at[dst_v.at[j]], add=True)
            pltpu.make_async_copy(
                h_hbm.at[src_v.at[j + 1]], gbuf1, sem1).wait()

            @pl.when(jj < NBH // 2 - 1)
            def _():
                pltpu.async_copy(h_hbm.at[src_v.at[j + 2]], gbuf0, sem0)

            pltpu.sync_copy(gbuf1, agg_sh.at[dst_v.at[j + 1]], add=True)
            return 0

        lax.fori_loop(0, NBH // 2, body, 0)
    plsc.subcore_barrier()
    _sliced_rows(s, lambda o, r: pltpu.sync_copy(
        agg_sh.at[pl.ds(o, r)], out_hbm.at[c, pl.ds(o, r)]))


def _prep_body(hist_ref, x_ref, xs_ref, ns_ref, nd_ref):
    h = hist_ref[...]
    deg_o = (h[0, 0] + h[1, 0])[:, 0:1]
    deg_i = (h[0, 1] + h[1, 1])[:, 0:1]
    ns = lax.rsqrt(jnp.maximum(deg_o, 1.0))
    nd = lax.rsqrt(jnp.maximum(deg_i, 1.0))
    xs_ref[...] = x_ref[...] * ns
    ns_ref[...] = ns
    nd_ref[...] = nd


def _prep_call(hist4, x):
    return pl.pallas_call(
        _prep_body,
        grid=(N // BN,),
        in_specs=[
            pl.BlockSpec((NCORE, 2, BN, 128), lambda i: (0, 0, i, 0)),
            pl.BlockSpec((BN, D_IN), lambda i: (i, 0)),
        ],
        out_specs=[
            pl.BlockSpec((BN, D_IN), lambda i: (i, 0)),
            pl.BlockSpec((BN, 1), lambda i: (i, 0)),
            pl.BlockSpec((BN, 1), lambda i: (i, 0)),
        ],
        out_shape=[
            jax.ShapeDtypeStruct((N, D_IN), jnp.float32),
            jax.ShapeDtypeStruct((N, 1), jnp.float32),
            jax.ShapeDtypeStruct((N, 1), jnp.float32),
        ],
    )(hist4, x)


def _mid_body(p_ref, w1_ref, b1_ref, ns_ref, nd_ref, w2_ref, h2_ref):
    p = p_ref[0] + p_ref[1]
    a = jnp.dot(p, w1_ref[...], preferred_element_type=jnp.float32)
    o1 = a * nd_ref[...] + b1_ref[...]
    r = jnp.maximum(o1, 0.0) * ns_ref[...]
    h2_ref[...] = jnp.dot(r, w2_ref[...], preferred_element_type=jnp.float32)


def _mid_call(p, W1, b1r, ns, nd, W2):
    return pl.pallas_call(
        _mid_body,
        grid=(N // BN,),
        in_specs=[
            pl.BlockSpec((NCORE, BN, 128), lambda i: (0, i, 0)),
            pl.BlockSpec((D_IN, D_H), lambda i: (0, 0)),
            pl.BlockSpec((1, D_H), lambda i: (0, 0)),
            pl.BlockSpec((BN, 1), lambda i: (i, 0)),
            pl.BlockSpec((BN, 1), lambda i: (i, 0)),
            pl.BlockSpec((D_H, D_OUT), lambda i: (0, 0)),
        ],
        out_specs=pl.BlockSpec((BN, D_OUT), lambda i: (i, 0)),
        out_shape=jax.ShapeDtypeStruct((N, D_OUT), jnp.float32),
    )(p, W1, b1r, ns, nd, W2)


def _fin_body(q_ref, nd_ref, b2_ref, o_ref):
    o_ref[...] = (q_ref[0] + q_ref[1]) * nd_ref[...] + b2_ref[...]


def _fin_call(q, nd, b2r):
    return pl.pallas_call(
        _fin_body,
        grid=(N // BN,),
        in_specs=[
            pl.BlockSpec((NCORE, BN, D_OUT), lambda i: (0, i, 0)),
            pl.BlockSpec((BN, 1), lambda i: (i, 0)),
            pl.BlockSpec((1, D_OUT), lambda i: (0, 0)),
        ],
        out_specs=pl.BlockSpec((BN, D_OUT), lambda i: (i, 0)),
        out_shape=jax.ShapeDtypeStruct((N, D_OUT), jnp.float32),
    )(q, nd, b2r)


def kernel(features, edge_index, W1, b1, W2, b2):
    ei = edge_index.astype(jnp.int32)
    src4 = ei[0].reshape(NW, NB, BB)
    dst4 = ei[1].reshape(NW, NB, BB)
    ones128 = jnp.ones((BB, 128), jnp.float32)
    zmat = jnp.zeros((N, 128), jnp.float32)

    hist4 = _deg_kernel(src4, dst4, ones128, zmat)
    xs, ns, nd = _prep_call(hist4, features)
    p = _spmm_kernel(xs, src4, dst4, zmat)
    h2 = _mid_call(p, W1, b1.reshape(1, D_H), ns, nd, W2)
    q = _spmm_kernel(h2, src4, dst4, zmat)
    return _fin_call(q, nd, b2.reshape(1, D_OUT))


# speedup vs baseline: 9.7786x; 1.0369x over previous
import functools

import jax
import jax.numpy as jnp
from jax import lax
from jax.experimental import pallas as pl
from jax.experimental.pallas import tpu as pltpu
from jax.experimental.pallas import tpu_sc as plsc

N = 10000
E = 320000
D_IN = 128
D_H = 256
D_OUT = 128

NCORE = 2
NSUB = 16
NW = NCORE * NSUB
EPW = E // NW
NB = 80
BB = EPW // NB
NBH = NB // 2
RPT = N // NSUB
BN = 1000

_sc_mesh = plsc.VectorSubcoreMesh(core_axis_name="c", subcore_axis_name="s")


ROWS_A = 624
ROWS_LAST = N - ROWS_A * (NSUB - 1)


def _sliced_rows(s, fn):
    @pl.when(s < NSUB - 1)
    def _():
        fn(pl.multiple_of(s * ROWS_A, 8), ROWS_A)

    @pl.when(s == NSUB - 1)
    def _():
        fn(ROWS_A * (NSUB - 1), ROWS_LAST)


@functools.partial(
    pl.kernel,
    mesh=_sc_mesh,
    out_type=jax.ShapeDtypeStruct((NCORE, N, 128), jnp.float32),
    scratch_types=[
        pltpu.VMEM((NBH, BB), jnp.int32),
        pltpu.VMEM((NBH, BB), jnp.int32),
        pltpu.VMEM((BB, 128), jnp.float32),
        pltpu.VMEM((BB, 128), jnp.float32),
        pltpu.VMEM_SHARED((N, 128), jnp.float32),
    ],
)
def _deg_kernel(src_hbm, dst_hbm, vconst_hbm, zmat_hbm, out_hbm,
                src_v, dst_v, vs, vd, hist):
    c = lax.axis_index("c")
    s = lax.axis_index("s")
    w = c * NSUB + s
    pltpu.sync_copy(vconst_hbm.at[0], vs)
    pltpu.sync_copy(vconst_hbm.at[1], vd)
    _sliced_rows(s, lambda o, r: pltpu.sync_copy(
        zmat_hbm.at[pl.ds(o, r)], hist.at[pl.ds(o, r)]))
    plsc.subcore_barrier()
    for h in range(2):
        pltpu.sync_copy(src_hbm.at[w, pl.ds(h * NBH, NBH)], src_v)
        pltpu.sync_copy(dst_hbm.at[w, pl.ds(h * NBH, NBH)], dst_v)

        def body(j, _):
            pltpu.sync_copy(vs, hist.at[src_v.at[j]], add=True)
            pltpu.sync_copy(vd, hist.at[dst_v.at[j]], add=True)
            return 0

        lax.fori_loop(0, NBH, body, 0)
    plsc.subcore_barrier()
    _sliced_rows(s, lambda o, r: pltpu.sync_copy(
        hist.at[pl.ds(o, r)], out_hbm.at[c, pl.ds(o, r)]))


@functools.partial(
    pl.kernel,
    mesh=_sc_mesh,
    out_type=jax.ShapeDtypeStruct((NCORE, N, 128), jnp.float32),
    scratch_types=[
        pltpu.VMEM((NBH, BB), jnp.int32),
        pltpu.VMEM((NBH, BB), jnp.int32),
        pltpu.VMEM((BB, 128), jnp.float32),
        pltpu.VMEM((BB, 128), jnp.float32),
        pltpu.VMEM_SHARED((N, 128), jnp.float32),
        pltpu.SemaphoreType.DMA,
        pltpu.SemaphoreType.DMA,
    ],
)
def _spmm_kernel(h_hbm, src_hbm, dst_hbm, zmat_hbm, out_hbm,
                 src_v, dst_v, gbuf0, gbuf1, agg_sh, sem0, sem1):
    c = lax.axis_index("c")
    s = lax.axis_index("s")
    w = c * NSUB + s
    _sliced_rows(s, lambda o, r: pltpu.sync_copy(
        zmat_hbm.at[pl.ds(o, r)], agg_sh.at[pl.ds(o, r)]))
    plsc.subcore_barrier()

    for h in range(2):
        pltpu.sync_copy(src_hbm.at[w, pl.ds(h * NBH, NBH)], src_v)
        pltpu.sync_copy(dst_hbm.at[w, pl.ds(h * NBH, NBH)], dst_v)
        pltpu.async_copy(h_hbm.at[src_v.at[0]], gbuf0, sem0)

        def body(jj, _):
            j = 2 * jj
            pltpu.make_async_copy(h_hbm.at[src_v.at[j]], gbuf0, sem0).wait()
            pltpu.async_copy(h_hbm.at[src_v.at[j + 1]], gbuf1, sem1)
            pltpu.sync_copy(gbuf0, agg_sh.at[dst_v.at[j]], add=True)
            pltpu.make_async_copy(
                h_hbm.at[src_v.at[j + 1]], gbuf1, sem1).wait()

            @pl.when(jj < NBH // 2 - 1)
            def _():
                pltpu.async_copy(h_hbm.at[src_v.at[j + 2]], gbuf0, sem0)

            pltpu.sync_copy(gbuf1, agg_sh.at[dst_v.at[j + 1]], add=True)
            return 0

        lax.fori_loop(0, NBH // 2, body, 0)
    plsc.subcore_barrier()
    _sliced_rows(s, lambda o, r: pltpu.sync_copy(
        agg_sh.at[pl.ds(o, r)], out_hbm.at[c, pl.ds(o, r)]))


def _prep_body(hist_ref, x_ref, xs_ref, ns_ref, nd_ref):
    h = hist_ref[...]
    deg_o = (h[0] + h[1])[:, 0:1]
    deg_i = (h[0] + h[1])[:, 8:9]
    ns = lax.rsqrt(jnp.maximum(deg_o, 1.0))
    nd = lax.rsqrt(jnp.maximum(deg_i, 1.0))
    xs_ref[...] = x_ref[...] * ns
    ns_ref[...] = ns
    nd_ref[...] = nd


def _prep_call(hist3, x):
    return pl.pallas_call(
        _prep_body,
        grid=(N // BN,),
        in_specs=[
            pl.BlockSpec((NCORE, BN, 128), lambda i: (0, i, 0)),
            pl.BlockSpec((BN, D_IN), lambda i: (i, 0)),
        ],
        out_specs=[
            pl.BlockSpec((BN, D_IN), lambda i: (i, 0)),
            pl.BlockSpec((BN, 1), lambda i: (i, 0)),
            pl.BlockSpec((BN, 1), lambda i: (i, 0)),
        ],
        out_shape=[
            jax.ShapeDtypeStruct((N, D_IN), jnp.float32),
            jax.ShapeDtypeStruct((N, 1), jnp.float32),
            jax.ShapeDtypeStruct((N, 1), jnp.float32),
        ],
    )(hist3, x)


def _mid_body(p_ref, w1_ref, b1_ref, ns_ref, nd_ref, w2_ref, h2_ref):
    p = p_ref[0] + p_ref[1]
    a = jnp.dot(p, w1_ref[...], preferred_element_type=jnp.float32)
    o1 = a * nd_ref[...] + b1_ref[...]
    r = jnp.maximum(o1, 0.0) * ns_ref[...]
    h2_ref[...] = jnp.dot(r, w2_ref[...], preferred_element_type=jnp.float32)


def _mid_call(p, W1, b1r, ns, nd, W2):
    return pl.pallas_call(
        _mid_body,
        grid=(N // BN,),
        in_specs=[
            pl.BlockSpec((NCORE, BN, 128), lambda i: (0, i, 0)),
            pl.BlockSpec((D_IN, D_H), lambda i: (0, 0)),
            pl.BlockSpec((1, D_H), lambda i: (0, 0)),
            pl.BlockSpec((BN, 1), lambda i: (i, 0)),
            pl.BlockSpec((BN, 1), lambda i: (i, 0)),
            pl.BlockSpec((D_H, D_OUT), lambda i: (0, 0)),
        ],
        out_specs=pl.BlockSpec((BN, D_OUT), lambda i: (i, 0)),
        out_shape=jax.ShapeDtypeStruct((N, D_OUT), jnp.float32),
    )(p, W1, b1r, ns, nd, W2)


def _fin_body(q_ref, nd_ref, b2_ref, o_ref):
    o_ref[...] = (q_ref[0] + q_ref[1]) * nd_ref[...] + b2_ref[...]


def _fin_call(q, nd, b2r):
    return pl.pallas_call(
        _fin_body,
        grid=(N // BN,),
        in_specs=[
            pl.BlockSpec((NCORE, BN, D_OUT), lambda i: (0, i, 0)),
            pl.BlockSpec((BN, 1), lambda i: (i, 0)),
            pl.BlockSpec((1, D_OUT), lambda i: (0, 0)),
        ],
        out_specs=pl.BlockSpec((BN, D_OUT), lambda i: (i, 0)),
        out_shape=jax.ShapeDtypeStruct((N, D_OUT), jnp.float32),
    )(q, nd, b2r)


def kernel(features, edge_index, W1, b1, W2, b2):
    ei = edge_index.astype(jnp.int32)
    src4 = ei[0].reshape(NW, NB, BB)
    dst4 = ei[1].reshape(NW, NB, BB)
    lane = lax.broadcasted_iota(jnp.int32, (2, BB, 128), 2)
    half = lax.broadcasted_iota(jnp.int32, (2, BB, 128), 0)
    vconst = jnp.where((lane // 8) == half, 1.0, 0.0).astype(jnp.float32)
    zmat = jnp.zeros((N, 128), jnp.float32)

    hist3 = _deg_kernel(src4, dst4, vconst, zmat)
    xs, ns, nd = _prep_call(hist3, features)
    p = _spmm_kernel(xs, src4, dst4, zmat)
    h2 = _mid_call(p, W1, b1.reshape(1, D_H), ns, nd, W2)
    q = _spmm_kernel(h2, src4, dst4, zmat)
    return _fin_call(q, nd, b2.reshape(1, D_OUT))
